# baseline (speedup 1.0000x reference)
import functools

import jax
import jax.numpy as jnp
from jax import lax
from jax.experimental import pallas as pl
from jax.experimental.pallas import tpu as pltpu
from jax.experimental.pallas import tpu_sc as plsc

NUM_GRAPHS = 64
CHUNK = 128
NUM_WORKERS = 32
BLK = 1000


@functools.partial(jax.jit, static_argnames=("n_pad",))
def _sc_aggregate(h, ei, n_pad):
    N, D = h.shape
    E = ei.shape[1]
    total_chunks = E // CHUNK
    nc = (total_chunks // NUM_WORKERS) & ~1
    rem = total_chunks - NUM_WORKERS * nc
    assert 0 <= rem <= NUM_WORKERS and nc % 2 == 0
    rows_pad = n_pad // 16
    rows_out = (N // 16) & ~7
    rows_rem = N - 16 * rows_out
    nzc = rows_pad // CHUNK
    mesh = plsc.VectorSubcoreMesh(core_axis_name="c", subcore_axis_name="s",
                                  num_cores=2, num_subcores=16)

    @functools.partial(
        pl.kernel,
        out_type=jax.ShapeDtypeStruct((2, N, D), jnp.float32),
        mesh=mesh,
        scratch_types=[
            pltpu.VMEM(((nc + 1) * CHUNK,), jnp.int32),
            pltpu.VMEM((CHUNK,), jnp.int32),
            pltpu.VMEM((CHUNK,), jnp.int32),
            pltpu.VMEM((CHUNK, D), jnp.float32),
            pltpu.VMEM((CHUNK, D), jnp.float32),
            pltpu.VMEM_SHARED((n_pad, D), jnp.float32),
            pltpu.SemaphoreType.DMA,
            pltpu.SemaphoreType.DMA,
            pltpu.SemaphoreType.DMA,
            pltpu.SemaphoreType.DMA,
        ],
    )
    def agg_kernel(h_hbm, ei_hbm, out_hbm,
                   sidx, didx0, didx1, rows0, rows1, acc, g0, g1, d0, d1):
        c = lax.axis_index("c")
        s = lax.axis_index("s")
        wid = c * 16 + s
        ebase = wid * (nc * CHUNK)

        pltpu.async_copy(ei_hbm.at[0, pl.ds(ebase, nc * CHUNK)],
                         sidx.at[pl.ds(0, nc * CHUNK)], g1)

        def zbody(r, carry):
            for j in range(D // 16):
                rows0[r, pl.ds(j * 16, 16)] = jnp.zeros((16,), jnp.float32)
            return carry
        lax.fori_loop(0, CHUNK, zbody, 0)
        zbase = s * rows_pad
        for j in range(nzc):
            pltpu.async_copy(rows0, acc.at[pl.ds(zbase + j * CHUNK, CHUNK)],
                             d0 if j % 2 == 0 else d1)
        pltpu.make_async_copy(ei_hbm.at[0, pl.ds(0, nc * CHUNK)],
                              sidx.at[pl.ds(0, nc * CHUNK)], g1).wait()
        for j in range(nzc):
            pltpu.make_async_copy(rows0,
                                  acc.at[pl.ds(zbase + j * CHUNK, CHUNK)],
                                  d0 if j % 2 == 0 else d1).wait()
        plsc.subcore_barrier()

        def sslc(k):
            return sidx.at[pl.ds(k * CHUNK, CHUNK)]

        def dslc(k):
            return ei_hbm.at[1, pl.ds(ebase + k * CHUNK, CHUNK)]

        pltpu.async_copy(dslc(0), didx0, d0)
        pltpu.async_copy(dslc(1), didx1, d1)
        pltpu.async_copy(h_hbm.at[sslc(0)], rows0, g0)

        def lbody(i, carry):
            k = i * 2
            pltpu.async_copy(h_hbm.at[sslc(k + 1)], rows1, g1)
            pltpu.make_async_copy(h_hbm.at[sslc(k)], rows0, g0).wait()
            pltpu.make_async_copy(dslc(0), didx0, d0).wait()
            pltpu.sync_copy(rows0, acc.at[didx0], add=True)
            nxt = jnp.minimum(k + 2, nc - 1)
            pltpu.async_copy(dslc(nxt), didx0, d0)
            pltpu.async_copy(h_hbm.at[sslc(nxt)], rows0, g0)
            pltpu.make_async_copy(h_hbm.at[sslc(k + 1)], rows1, g1).wait()
            pltpu.make_async_copy(dslc(0), didx1, d1).wait()
            pltpu.sync_copy(rows1, acc.at[didx1], add=True)
            nxt1 = jnp.minimum(k + 3, nc - 1)
            pltpu.async_copy(dslc(nxt1), didx1, d1)
            return carry
        lax.fori_loop(0, nc // 2, lbody, 0)
        pltpu.make_async_copy(h_hbm.at[sslc(nc - 1)], rows0, g0).wait()
        pltpu.make_async_copy(dslc(0), didx0, d0).wait()
        pltpu.make_async_copy(dslc(0), didx1, d1).wait()

        if rem:
            @pl.when(wid < rem)
            def _():
                tb = (NUM_WORKERS * nc + wid) * CHUNK
                pltpu.sync_copy(ei_hbm.at[0, pl.ds(tb, CHUNK)],
                                sidx.at[pl.ds(nc * CHUNK, CHUNK)])
                pltpu.async_copy(ei_hbm.at[1, pl.ds(tb, CHUNK)], didx0, d0)
                pltpu.async_copy(h_hbm.at[sslc(nc)], rows0, g0)
                pltpu.make_async_copy(h_hbm.at[sslc(nc)], rows0, g0).wait()
                pltpu.make_async_copy(dslc(0), didx0, d0).wait()
                pltpu.sync_copy(rows0, acc.at[didx0], add=True)
        plsc.subcore_barrier()

        ob = s * rows_out
        pltpu.sync_copy(acc.at[pl.ds(ob, rows_out)],
                        out_hbm.at[c, pl.ds(ob, rows_out)])
        if rows_rem:
            @pl.when(s == 15)
            def _():
                rb = 16 * rows_out
                pltpu.sync_copy(acc.at[pl.ds(rb, rows_rem)],
                                out_hbm.at[c, pl.ds(rb, rows_rem)])

    return agg_kernel(h, ei)


def _layer_tc(h, agg, batch_r, w1, b1, w2, b2, with_x_readout):
    N, D = h.shape
    IH = w1.shape[1]
    DO = w2.shape[1]
    nb = N // BLK

    def body(h_ref, a_ref, b_ref, w1_ref, b1_ref, w2_ref, b2_ref,
             hn_ref, ro_ref, *maybe_rox):
        i = pl.program_id(0)
        z = h_ref[...] + a_ref[0] + a_ref[1]
        hm = jnp.maximum(
            jnp.dot(z, w1_ref[...], preferred_element_type=jnp.float32)
            + b1_ref[...], 0.0)
        hn = (jnp.dot(hm, w2_ref[...], preferred_element_type=jnp.float32)
              + b2_ref[...])
        hn_ref[...] = hn
        bvec = b_ref[0, 0, :]
        onehot = (bvec[None, :] == lax.broadcasted_iota(
            jnp.int32, (NUM_GRAPHS, BLK), 0)).astype(jnp.float32)

        @pl.when(i == 0)
        def _():
            ro_ref[...] = jnp.zeros_like(ro_ref)
        ro_ref[...] += jnp.dot(onehot, hn,
                               preferred_element_type=jnp.float32)
        if maybe_rox:
            rox_ref = maybe_rox[0]

            @pl.when(i == 0)
            def _():
                rox_ref[...] = jnp.zeros_like(rox_ref)
            rox_ref[...] += jnp.dot(onehot, h_ref[...],
                                    preferred_element_type=jnp.float32)

    out_shape = [jax.ShapeDtypeStruct((N, DO), jnp.float32),
                 jax.ShapeDtypeStruct((NUM_GRAPHS, DO), jnp.float32)]
    out_specs = [pl.BlockSpec((BLK, DO), lambda i: (i, 0)),
                 pl.BlockSpec((NUM_GRAPHS, DO), lambda i: (0, 0))]
    if with_x_readout:
        out_shape.append(jax.ShapeDtypeStruct((NUM_GRAPHS, D), jnp.float32))
        out_specs.append(pl.BlockSpec((NUM_GRAPHS, D), lambda i: (0, 0)))

    return pl.pallas_call(
        body,
        grid=(nb,),
        in_specs=[
            pl.BlockSpec((BLK, D), lambda i: (i, 0)),
            pl.BlockSpec((2, BLK, D), lambda i: (0, i, 0)),
            pl.BlockSpec((1, 1, BLK), lambda i: (i, 0, 0)),
            pl.BlockSpec((D, IH), lambda i: (0, 0)),
            pl.BlockSpec((1, IH), lambda i: (0, 0)),
            pl.BlockSpec((IH, DO), lambda i: (0, 0)),
            pl.BlockSpec((1, DO), lambda i: (0, 0)),
        ],
        out_specs=out_specs,
        out_shape=out_shape,
    )(h, agg, batch_r, w1, b1.reshape(1, IH), w2, b2.reshape(1, DO))


def _final_tc(ro0, ro1, ro2, ro3, f_w1, f_b1, f_w2, f_b2):
    G, D = ro0.shape
    OI = f_w1.shape[1]
    OD = f_w2.shape[1]

    def body(r0, r1, r2, r3, w10, w11, w12, w13, fb1, fw2, fb2, out):
        z = (jnp.dot(r0[...], w10[...], preferred_element_type=jnp.float32)
             + jnp.dot(r1[...], w11[...], preferred_element_type=jnp.float32)
             + jnp.dot(r2[...], w12[...], preferred_element_type=jnp.float32)
             + jnp.dot(r3[...], w13[...], preferred_element_type=jnp.float32)
             + fb1[...])
        z = jnp.maximum(z, 0.0)
        out[...] = (jnp.dot(z, fw2[...], preferred_element_type=jnp.float32)
                    + fb2[...])

    return pl.pallas_call(
        body,
        out_shape=jax.ShapeDtypeStruct((G, OD), jnp.float32),
    )(ro0, ro1, ro2, ro3,
      f_w1[0 * D:1 * D], f_w1[1 * D:2 * D], f_w1[2 * D:3 * D],
      f_w1[3 * D:4 * D],
      f_b1.reshape(1, OI), f_w2, f_b2.reshape(1, OD))


def kernel(x, edge_index, batch,
           l0_w1, l0_b1, l0_w2, l0_b2,
           l1_w1, l1_b1, l1_w2, l1_b2,
           l2_w1, l2_b1, l2_w2, l2_b2,
           f_w1, f_b1, f_w2, f_b2):
    N, D = x.shape
    E = edge_index.shape[1]

    n_pad = -(-(N + 1) // (16 * CHUNK)) * (16 * CHUNK)
    ei = edge_index.astype(jnp.int32)
    if E % CHUNK:
        e_pad = -(-E // CHUNK) * CHUNK
        pad_ar = jnp.arange(e_pad - E, dtype=jnp.int32)
        pad = jnp.stack([pad_ar % N, N + pad_ar % (n_pad - N)])
        ei = jnp.concatenate([ei, pad], axis=1)

    nb = N // BLK
    batch_r = batch.astype(jnp.int32).reshape(nb, 1, BLK)

    agg = _sc_aggregate(x, ei, n_pad=n_pad)
    h1, ro1, ro0 = _layer_tc(x, agg, batch_r,
                             l0_w1, l0_b1, l0_w2, l0_b2, True)
    agg = _sc_aggregate(h1, ei, n_pad=n_pad)
    h2, ro2 = _layer_tc(h1, agg, batch_r,
                        l1_w1, l1_b1, l1_w2, l1_b2, False)
    agg = _sc_aggregate(h2, ei, n_pad=n_pad)
    h3, ro3 = _layer_tc(h2, agg, batch_r,
                        l2_w1, l2_b1, l2_w2, l2_b2, False)
    return _final_tc(ro0, ro1, ro2, ro3, f_w1, f_b1, f_w2, f_b2)


# speedup vs baseline: 1.0264x; 1.0264x over previous
import functools

import jax
import jax.numpy as jnp
from jax import lax
from jax.experimental import pallas as pl
from jax.experimental.pallas import tpu as pltpu
from jax.experimental.pallas import tpu_sc as plsc

NUM_GRAPHS = 64
CHUNK = 128
NUM_WORKERS = 32
BLK = 2000


@functools.partial(jax.jit, static_argnames=("n_pad",))
def _sc_aggregate(h, ei, n_pad):
    N, D = h.shape
    E = ei.shape[1]
    total_chunks = E // CHUNK
    nc = (total_chunks // NUM_WORKERS) & ~1
    rem = total_chunks - NUM_WORKERS * nc
    assert 0 <= rem <= NUM_WORKERS and nc % 2 == 0
    rows_pad = n_pad // 16
    rows_out = (N // 16) & ~7
    rows_rem = N - 16 * rows_out
    nzc = rows_pad // CHUNK
    mesh = plsc.VectorSubcoreMesh(core_axis_name="c", subcore_axis_name="s",
                                  num_cores=2, num_subcores=16)

    @functools.partial(
        pl.kernel,
        out_type=jax.ShapeDtypeStruct((2, N, D), jnp.float32),
        mesh=mesh,
        scratch_types=[
            pltpu.VMEM(((nc + 1) * CHUNK,), jnp.int32),
            pltpu.VMEM((CHUNK,), jnp.int32),
            pltpu.VMEM((CHUNK,), jnp.int32),
            pltpu.VMEM((CHUNK, D), jnp.float32),
            pltpu.VMEM((CHUNK, D), jnp.float32),
            pltpu.VMEM_SHARED((n_pad, D), jnp.float32),
            pltpu.SemaphoreType.DMA,
            pltpu.SemaphoreType.DMA,
            pltpu.SemaphoreType.DMA,
            pltpu.SemaphoreType.DMA,
        ],
    )
    def agg_kernel(h_hbm, ei_hbm, out_hbm,
                   sidx, didx0, didx1, rows0, rows1, acc, g0, g1, d0, d1):
        c = lax.axis_index("c")
        s = lax.axis_index("s")
        wid = c * 16 + s
        ebase = wid * (nc * CHUNK)

        pltpu.async_copy(ei_hbm.at[0, pl.ds(ebase, nc * CHUNK)],
                         sidx.at[pl.ds(0, nc * CHUNK)], g1)

        def zbody(r, carry):
            for j in range(D // 16):
                rows0[r, pl.ds(j * 16, 16)] = jnp.zeros((16,), jnp.float32)
            return carry
        lax.fori_loop(0, CHUNK, zbody, 0)
        zbase = s * rows_pad
        for j in range(nzc):
            pltpu.async_copy(rows0, acc.at[pl.ds(zbase + j * CHUNK, CHUNK)],
                             d0 if j % 2 == 0 else d1)
        pltpu.make_async_copy(ei_hbm.at[0, pl.ds(0, nc * CHUNK)],
                              sidx.at[pl.ds(0, nc * CHUNK)], g1).wait()
        for j in range(nzc):
            pltpu.make_async_copy(rows0,
                                  acc.at[pl.ds(zbase + j * CHUNK, CHUNK)],
                                  d0 if j % 2 == 0 else d1).wait()
        plsc.subcore_barrier()

        def sslc(k):
            return sidx.at[pl.ds(k * CHUNK, CHUNK)]

        def dslc(k):
            return ei_hbm.at[1, pl.ds(ebase + k * CHUNK, CHUNK)]

        pltpu.async_copy(dslc(0), didx0, d0)
        pltpu.async_copy(dslc(1), didx1, d1)
        pltpu.async_copy(h_hbm.at[sslc(0)], rows0, g0)

        def lbody(i, carry):
            k = i * 2
            pltpu.async_copy(h_hbm.at[sslc(k + 1)], rows1, g1)
            pltpu.make_async_copy(h_hbm.at[sslc(k)], rows0, g0).wait()
            pltpu.make_async_copy(dslc(0), didx0, d0).wait()
            pltpu.sync_copy(rows0, acc.at[didx0], add=True)
            nxt = jnp.minimum(k + 2, nc - 1)
            pltpu.async_copy(dslc(nxt), didx0, d0)
            pltpu.async_copy(h_hbm.at[sslc(nxt)], rows0, g0)
            pltpu.make_async_copy(h_hbm.at[sslc(k + 1)], rows1, g1).wait()
            pltpu.make_async_copy(dslc(0), didx1, d1).wait()
            pltpu.sync_copy(rows1, acc.at[didx1], add=True)
            nxt1 = jnp.minimum(k + 3, nc - 1)
            pltpu.async_copy(dslc(nxt1), didx1, d1)
            return carry
        lax.fori_loop(0, nc // 2, lbody, 0)
        pltpu.make_async_copy(h_hbm.at[sslc(nc - 1)], rows0, g0).wait()
        pltpu.make_async_copy(dslc(0), didx0, d0).wait()
        pltpu.make_async_copy(dslc(0), didx1, d1).wait()

        if rem:
            @pl.when(wid < rem)
            def _():
                tb = (NUM_WORKERS * nc + wid) * CHUNK
                pltpu.sync_copy(ei_hbm.at[0, pl.ds(tb, CHUNK)],
                                sidx.at[pl.ds(nc * CHUNK, CHUNK)])
                pltpu.async_copy(ei_hbm.at[1, pl.ds(tb, CHUNK)], didx0, d0)
                pltpu.async_copy(h_hbm.at[sslc(nc)], rows0, g0)
                pltpu.make_async_copy(h_hbm.at[sslc(nc)], rows0, g0).wait()
                pltpu.make_async_copy(dslc(0), didx0, d0).wait()
                pltpu.sync_copy(rows0, acc.at[didx0], add=True)
        plsc.subcore_barrier()

        ob = s * rows_out
        pltpu.sync_copy(acc.at[pl.ds(ob, rows_out)],
                        out_hbm.at[c, pl.ds(ob, rows_out)])
        if rows_rem:
            @pl.when(s == 15)
            def _():
                rb = 16 * rows_out
                pltpu.sync_copy(acc.at[pl.ds(rb, rows_rem)],
                                out_hbm.at[c, pl.ds(rb, rows_rem)])

    return agg_kernel(h, ei)


def _layer_tc(h, agg, batch_r, w1, b1, w2, b2, with_x_readout):
    N, D = h.shape
    IH = w1.shape[1]
    DO = w2.shape[1]
    nb = N // BLK

    def body(h_ref, a_ref, b_ref, w1_ref, b1_ref, w2_ref, b2_ref,
             hn_ref, ro_ref, *maybe_rox):
        i = pl.program_id(0)
        z = h_ref[...] + a_ref[0] + a_ref[1]
        hm = jnp.maximum(
            jnp.dot(z, w1_ref[...], preferred_element_type=jnp.float32)
            + b1_ref[...], 0.0)
        hn = (jnp.dot(hm, w2_ref[...], preferred_element_type=jnp.float32)
              + b2_ref[...])
        hn_ref[...] = hn
        bvec = b_ref[0, 0, :]
        onehot = (bvec[None, :] == lax.broadcasted_iota(
            jnp.int32, (NUM_GRAPHS, BLK), 0)).astype(jnp.float32)

        @pl.when(i == 0)
        def _():
            ro_ref[...] = jnp.zeros_like(ro_ref)
        ro_ref[...] += jnp.dot(onehot, hn,
                               preferred_element_type=jnp.float32)
        if maybe_rox:
            rox_ref = maybe_rox[0]

            @pl.when(i == 0)
            def _():
                rox_ref[...] = jnp.zeros_like(rox_ref)
            rox_ref[...] += jnp.dot(onehot, h_ref[...],
                                    preferred_element_type=jnp.float32)

    out_shape = [jax.ShapeDtypeStruct((N, DO), jnp.float32),
                 jax.ShapeDtypeStruct((NUM_GRAPHS, DO), jnp.float32)]
    out_specs = [pl.BlockSpec((BLK, DO), lambda i: (i, 0)),
                 pl.BlockSpec((NUM_GRAPHS, DO), lambda i: (0, 0))]
    if with_x_readout:
        out_shape.append(jax.ShapeDtypeStruct((NUM_GRAPHS, D), jnp.float32))
        out_specs.append(pl.BlockSpec((NUM_GRAPHS, D), lambda i: (0, 0)))

    return pl.pallas_call(
        body,
        grid=(nb,),
        in_specs=[
            pl.BlockSpec((BLK, D), lambda i: (i, 0)),
            pl.BlockSpec((2, BLK, D), lambda i: (0, i, 0)),
            pl.BlockSpec((1, 1, BLK), lambda i: (i, 0, 0)),
            pl.BlockSpec((D, IH), lambda i: (0, 0)),
            pl.BlockSpec((1, IH), lambda i: (0, 0)),
            pl.BlockSpec((IH, DO), lambda i: (0, 0)),
            pl.BlockSpec((1, DO), lambda i: (0, 0)),
        ],
        out_specs=out_specs,
        out_shape=out_shape,
    )(h, agg, batch_r, w1, b1.reshape(1, IH), w2, b2.reshape(1, DO))


def _final_tc(ro0, ro1, ro2, ro3, f_w1, f_b1, f_w2, f_b2):
    G, D = ro0.shape
    OI = f_w1.shape[1]
    OD = f_w2.shape[1]

    def body(r0, r1, r2, r3, w10, w11, w12, w13, fb1, fw2, fb2, out):
        z = (jnp.dot(r0[...], w10[...], preferred_element_type=jnp.float32)
             + jnp.dot(r1[...], w11[...], preferred_element_type=jnp.float32)
             + jnp.dot(r2[...], w12[...], preferred_element_type=jnp.float32)
             + jnp.dot(r3[...], w13[...], preferred_element_type=jnp.float32)
             + fb1[...])
        z = jnp.maximum(z, 0.0)
        out[...] = (jnp.dot(z, fw2[...], preferred_element_type=jnp.float32)
                    + fb2[...])

    return pl.pallas_call(
        body,
        out_shape=jax.ShapeDtypeStruct((G, OD), jnp.float32),
    )(ro0, ro1, ro2, ro3,
      f_w1[0 * D:1 * D], f_w1[1 * D:2 * D], f_w1[2 * D:3 * D],
      f_w1[3 * D:4 * D],
      f_b1.reshape(1, OI), f_w2, f_b2.reshape(1, OD))


def kernel(x, edge_index, batch,
           l0_w1, l0_b1, l0_w2, l0_b2,
           l1_w1, l1_b1, l1_w2, l1_b2,
           l2_w1, l2_b1, l2_w2, l2_b2,
           f_w1, f_b1, f_w2, f_b2):
    N, D = x.shape
    E = edge_index.shape[1]

    n_pad = -(-(N + 1) // (16 * CHUNK)) * (16 * CHUNK)
    ei = edge_index.astype(jnp.int32)
    if E % CHUNK:
        e_pad = -(-E // CHUNK) * CHUNK
        pad_ar = jnp.arange(e_pad - E, dtype=jnp.int32)
        pad = jnp.stack([pad_ar % N, N + pad_ar % (n_pad - N)])
        ei = jnp.concatenate([ei, pad], axis=1)

    nb = N // BLK
    batch_r = batch.astype(jnp.int32).reshape(nb, 1, BLK)

    agg = _sc_aggregate(x, ei, n_pad=n_pad)
    h1, ro1, ro0 = _layer_tc(x, agg, batch_r,
                             l0_w1, l0_b1, l0_w2, l0_b2, True)
    agg = _sc_aggregate(h1, ei, n_pad=n_pad)
    h2, ro2 = _layer_tc(h1, agg, batch_r,
                        l1_w1, l1_b1, l1_w2, l1_b2, False)
    agg = _sc_aggregate(h2, ei, n_pad=n_pad)
    h3, ro3 = _layer_tc(h2, agg, batch_r,
                        l2_w1, l2_b1, l2_w2, l2_b2, False)
    return _final_tc(ro0, ro1, ro2, ro3, f_w1, f_b1, f_w2, f_b2)
